# baseline (speedup 1.0000x reference)
import functools

import jax
import jax.numpy as jnp
from jax import lax
from jax.experimental import pallas as pl
from jax.experimental.pallas import tpu as pltpu
from jax.experimental.pallas import tpu_sc as plsc

LANES = 16
NC = 2
NS = 16
NW = NC * NS
NBUF = 2
S = 2


@functools.lru_cache(maxsize=None)
def _build(n_b, n_l, vocab, emb, n_pos, pos_w, n_bio, bio_w):
    out_w = emb + pos_w + bio_w
    pb_w = pos_w + bio_w
    chunk = S * n_l
    assert n_b % (NW * S * NBUF) == 0 and chunk % LANES == 0
    sent_per_w = n_b // NW
    n_chunks = sent_per_w // S

    mesh = plsc.VectorSubcoreMesh(core_axis_name="c", subcore_axis_name="s")

    per_buf = [
        pltpu.VMEM((S, n_l), jnp.int32),
        pltpu.VMEM((S, n_l, 2), jnp.int32),
        pltpu.VMEM((S, n_l, emb), jnp.float32),
        pltpu.VMEM((S, n_l, pb_w), jnp.float32),
        pltpu.SemaphoreType.DMA,
        pltpu.SemaphoreType.DMA,
        pltpu.SemaphoreType.DMA,
    ]

    @functools.partial(
        pl.kernel,
        mesh=mesh,
        out_type=jax.ShapeDtypeStruct((n_b, n_l, out_w), jnp.float32),
        compiler_params=pltpu.CompilerParams(
            needs_layout_passes=False, use_tc_tiling_on_sc=False),
        scratch_types=[
            pltpu.VMEM((n_pos * pos_w,), jnp.float32),
            pltpu.VMEM((n_bio * bio_w,), jnp.float32),
        ] + per_buf * NBUF,
    )
    def sc_kernel(sent_hbm, feat_hbm, pos_hbm, bio_hbm, table_hbm,
                  out_hbm, pos_v, bio_v, *bufs):
        wid = lax.axis_index("s") * NC + lax.axis_index("c")
        base = wid * sent_per_w
        pltpu.sync_copy(pos_hbm, pos_v)
        pltpu.sync_copy(bio_hbm, bio_v)
        iota = lax.iota(jnp.int32, LANES)
        nb = len(per_buf)

        def do_chunk(g, b, drain):
            idx_v, feat_v, word_v, pb_v, gsem, wsem, psem = (
                bufs[b * nb:(b + 1) * nb])
            s0 = base + (g * NBUF + b) * S
            word_dst = out_hbm.at[pl.ds(s0, S), :, pl.ds(0, emb)]
            pb_dst = out_hbm.at[pl.ds(s0, S), :, pl.ds(emb, pb_w)]

            @pl.when(drain)
            def _():
                pltpu.make_async_copy(word_v, word_dst, wsem).wait()
                pltpu.make_async_copy(pb_v, pb_dst, psem).wait()

            pltpu.sync_copy(sent_hbm.at[pl.ds(s0, S)], idx_v)
            gathers = []
            for s in range(S):
                gathers.append(pltpu.async_copy(
                    table_hbm.at[idx_v.at[s]], word_v.at[s], gsem))
            pltpu.sync_copy(feat_hbm.at[pl.ds(s0, S)], feat_v)

            def fill_body(t, c):
                flat = t * LANES + iota
                sv = flat // n_l
                tv = flat % n_l
                zeros = jnp.zeros((LANES,), jnp.int32)
                f0s = plsc.load_gather(feat_v, [sv, tv, zeros]) * pos_w
                f1s = plsc.load_gather(feat_v, [sv, tv, zeros + 1]) * bio_w
                for col in range(pos_w):
                    vals = plsc.load_gather(pos_v, [f0s + col])
                    plsc.store_scatter(
                        pb_v, [sv, tv, zeros + col], vals)
                for col in range(bio_w):
                    vals = plsc.load_gather(bio_v, [f1s + col])
                    plsc.store_scatter(
                        pb_v, [sv, tv, zeros + (pos_w + col)], vals)
                return c

            lax.fori_loop(0, chunk // LANES, fill_body, 0)
            pltpu.async_copy(pb_v, pb_dst, psem)
            for cp in gathers:
                cp.wait()
            pltpu.async_copy(word_v, word_dst, wsem)

        def pair_body(g, carry):
            for b in range(NBUF):
                do_chunk(g, b, g > 0)
            return carry

        lax.fori_loop(0, n_chunks // NBUF, pair_body, 0)
        for b in range(NBUF):
            idx_v, feat_v, word_v, pb_v, gsem, wsem, psem = (
                bufs[b * nb:(b + 1) * nb])
            s0 = base + (n_chunks - NBUF + b) * S
            pltpu.make_async_copy(
                word_v, out_hbm.at[pl.ds(s0, S), :, pl.ds(0, emb)],
                wsem).wait()
            pltpu.make_async_copy(
                pb_v, out_hbm.at[pl.ds(s0, S), :, pl.ds(emb, pb_w)],
                psem).wait()

    return sc_kernel


def kernel(sentence, features, embedding_matrix, pos_table, bio_table):
    n_b, n_l = sentence.shape
    vocab, emb = embedding_matrix.shape
    n_pos, pos_w = pos_table.shape
    n_bio, bio_w = bio_table.shape
    fn = _build(n_b, n_l, vocab, emb, n_pos, pos_w, n_bio, bio_w)
    return fn(sentence, features, pos_table.reshape(n_pos * pos_w),
              bio_table.reshape(n_bio * bio_w), embedding_matrix)


# speedup vs baseline: 1.5343x; 1.5343x over previous
import functools

import jax
import jax.numpy as jnp
from jax import lax
from jax.experimental import pallas as pl
from jax.experimental.pallas import tpu as pltpu
from jax.experimental.pallas import tpu_sc as plsc

LANES = 16
NC = 2
NS = 16
NW = NC * NS
NBUF = 2


@functools.lru_cache(maxsize=None)
def _build(n_tok, vocab, emb, n_pb, pb_w):
    pb_pad = (pb_w + 7) // 8 * 8
    out_w = emb + pb_pad
    chunk = 512
    assert n_tok % (NW * chunk * NBUF) == 0
    tok_per_w = n_tok // NW
    n_chunks = tok_per_w // chunk

    mesh = plsc.VectorSubcoreMesh(core_axis_name="c", subcore_axis_name="s")

    per_buf = [
        pltpu.VMEM((chunk,), jnp.int32),
        pltpu.VMEM((chunk,), jnp.int32),
        pltpu.VMEM((chunk, emb), jnp.float32),
        pltpu.VMEM((chunk, pb_pad), jnp.float32),
        pltpu.SemaphoreType.DMA,
        pltpu.SemaphoreType.DMA,
        pltpu.SemaphoreType.DMA,
        pltpu.SemaphoreType.DMA,
    ]

    @functools.partial(
        pl.kernel,
        mesh=mesh,
        out_type=jax.ShapeDtypeStruct((n_tok, out_w), jnp.float32),
        compiler_params=pltpu.CompilerParams(
            needs_layout_passes=False, use_tc_tiling_on_sc=False),
        scratch_types=list(per_buf) * NBUF,
    )
    def sc_kernel(sent_hbm, code_hbm, pbtab_hbm, table_hbm, out_hbm,
                  *bufs):
        wid = lax.axis_index("s") * NC + lax.axis_index("c")
        base = wid * tok_per_w
        nb = len(per_buf)

        def do_chunk(g, b, drain):
            idx_v, code_v, word_v, pb_v, gsem, bsem, wsem, psem = (
                bufs[b * nb:(b + 1) * nb])
            cb = base + (g * NBUF + b) * chunk
            word_dst = out_hbm.at[pl.ds(cb, chunk), pl.ds(0, emb)]
            pb_dst = out_hbm.at[pl.ds(cb, chunk), pl.ds(emb, pb_pad)]

            @pl.when(drain)
            def _():
                pltpu.make_async_copy(word_v, word_dst, wsem).wait()
                pltpu.make_async_copy(pb_v, pb_dst, psem).wait()

            pltpu.sync_copy(sent_hbm.at[pl.ds(cb, chunk)], idx_v)
            wcp = pltpu.async_copy(table_hbm.at[idx_v], word_v, gsem)
            pltpu.sync_copy(code_hbm.at[pl.ds(cb, chunk)], code_v)
            bcp = pltpu.async_copy(pbtab_hbm.at[code_v], pb_v, bsem)
            bcp.wait()
            pltpu.async_copy(pb_v, pb_dst, psem)
            wcp.wait()
            pltpu.async_copy(word_v, word_dst, wsem)

        def pair_body(g, carry):
            for b in range(NBUF):
                do_chunk(g, b, g > 0)
            return carry

        lax.fori_loop(0, n_chunks // NBUF, pair_body, 0)
        for b in range(NBUF):
            idx_v, code_v, word_v, pb_v, gsem, bsem, wsem, psem = (
                bufs[b * nb:(b + 1) * nb])
            cb = base + (n_chunks - NBUF + b) * chunk
            pltpu.make_async_copy(
                word_v, out_hbm.at[pl.ds(cb, chunk), pl.ds(0, emb)],
                wsem).wait()
            pltpu.make_async_copy(
                pb_v,
                out_hbm.at[pl.ds(cb, chunk), pl.ds(emb, pb_pad)],
                psem).wait()

    return sc_kernel


def kernel(sentence, features, embedding_matrix, pos_table, bio_table):
    n_b, n_l = sentence.shape
    vocab, emb = embedding_matrix.shape
    n_pos, pos_w = pos_table.shape
    n_bio, bio_w = bio_table.shape
    n_tok = n_b * n_l
    sent = sentence.reshape(n_tok)
    code = (features[..., 0] * n_bio + features[..., 1]).reshape(n_tok)
    pb_w = pos_w + bio_w
    pb_pad = (pb_w + 7) // 8 * 8
    pbtab = jnp.concatenate(
        [jnp.repeat(pos_table, n_bio, axis=0),
         jnp.tile(bio_table, (n_pos, 1)),
         jnp.zeros((n_pos * n_bio, pb_pad - pb_w), jnp.float32)], axis=1)
    fn = _build(n_tok, vocab, emb, n_pos * n_bio, pb_w)
    out = fn(sent, code, pbtab, embedding_matrix)
    return out[:, :emb + pb_w].reshape(n_b, n_l, emb + pb_w)


# speedup vs baseline: 1.8532x; 1.2078x over previous
import functools

import jax
import jax.numpy as jnp
from jax import lax
from jax.experimental import pallas as pl
from jax.experimental.pallas import tpu as pltpu
from jax.experimental.pallas import tpu_sc as plsc

LANES = 16
NC = 2
NS = 16
NW = NC * NS
NBUF = 2
IDX_BITS = 20


@functools.lru_cache(maxsize=None)
def _build(n_tok, vocab, emb, n_pb, pb_w):
    out_w = emb + pb_w
    pb_pad = (pb_w + 7) // 8 * 8
    chunk = 512
    assert n_tok % (NW * chunk * NBUF) == 0 and chunk % 128 == 0
    tok_per_w = n_tok // NW
    n_chunks = tok_per_w // chunk

    mesh = plsc.VectorSubcoreMesh(core_axis_name="c", subcore_axis_name="s")

    per_buf = [
        pltpu.VMEM((chunk // 128, 128), jnp.int32),
        pltpu.VMEM((chunk,), jnp.int32),
        pltpu.VMEM((chunk,), jnp.int32),
        pltpu.VMEM((chunk, emb), jnp.float32),
        pltpu.VMEM((chunk, pb_w), jnp.float32),
        pltpu.SemaphoreType.DMA,
        pltpu.SemaphoreType.DMA,
        pltpu.SemaphoreType.DMA,
    ]

    @functools.partial(
        pl.kernel,
        mesh=mesh,
        out_type=jax.ShapeDtypeStruct((n_tok, out_w), jnp.float32),
        compiler_params=pltpu.CompilerParams(
            needs_layout_passes=False, use_tc_tiling_on_sc=False),
        scratch_types=[
            pltpu.VMEM((n_pb, pb_pad), jnp.float32),
        ] + list(per_buf) * NBUF,
    )
    def sc_kernel(packed_hbm, pbtab_hbm, table_hbm, out_hbm, pbtab_v,
                  *bufs):
        wid = lax.axis_index("s") * NC + lax.axis_index("c")
        base = wid * tok_per_w
        pltpu.sync_copy(pbtab_hbm, pbtab_v)
        iota = lax.iota(jnp.int32, LANES)
        nb = len(per_buf)

        def do_chunk(g, b, drain):
            pk_v, idx_v, code_v, word_v, pb_v, gsem, wsem, psem = (
                bufs[b * nb:(b + 1) * nb])
            cb = base + (g * NBUF + b) * chunk
            word_dst = out_hbm.at[pl.ds(cb, chunk), pl.ds(0, emb)]
            pb_dst = out_hbm.at[pl.ds(cb, chunk), pl.ds(emb, pb_w)]

            @pl.when(drain)
            def _():
                pltpu.make_async_copy(word_v, word_dst, wsem).wait()
                pltpu.make_async_copy(pb_v, pb_dst, psem).wait()

            pltpu.sync_copy(
                packed_hbm.at[pl.ds(cb // 128, chunk // 128)], pk_v)

            def unpack_body(t, c):
                r = t // (128 // LANES)
                col = (t % (128 // LANES)) * LANES
                pk = pk_v[r, pl.ds(col, LANES)]
                idx_v[pl.ds(t * LANES, LANES)] = pk & ((1 << IDX_BITS) - 1)
                code_v[pl.ds(t * LANES, LANES)] = (
                    lax.shift_right_logical(pk, IDX_BITS))
                return c

            lax.fori_loop(0, chunk // LANES, unpack_body, 0)
            wcp = pltpu.async_copy(table_hbm.at[idx_v], word_v, gsem)

            def fill_body(t, c):
                rows = t * LANES + iota
                codes = code_v[pl.ds(t * LANES, LANES)]
                for col in range(pb_w):
                    vals = plsc.load_gather(
                        pbtab_v,
                        [codes, jnp.full((LANES,), col, jnp.int32)])
                    plsc.store_scatter(
                        pb_v, [rows, jnp.full((LANES,), col, jnp.int32)],
                        vals)
                return c

            lax.fori_loop(0, chunk // LANES, fill_body, 0)
            pltpu.async_copy(pb_v, pb_dst, psem)
            wcp.wait()
            pltpu.async_copy(word_v, word_dst, wsem)

        def pair_body(g, carry):
            for b in range(NBUF):
                do_chunk(g, b, g > 0)
            return carry

        lax.fori_loop(0, n_chunks // NBUF, pair_body, 0)
        for b in range(NBUF):
            pk_v, idx_v, code_v, word_v, pb_v, gsem, wsem, psem = (
                bufs[b * nb:(b + 1) * nb])
            cb = base + (n_chunks - NBUF + b) * chunk
            pltpu.make_async_copy(
                word_v, out_hbm.at[pl.ds(cb, chunk), pl.ds(0, emb)],
                wsem).wait()
            pltpu.make_async_copy(
                pb_v, out_hbm.at[pl.ds(cb, chunk), pl.ds(emb, pb_w)],
                psem).wait()

    return sc_kernel


def kernel(sentence, features, embedding_matrix, pos_table, bio_table):
    n_b, n_l = sentence.shape
    vocab, emb = embedding_matrix.shape
    n_pos, pos_w = pos_table.shape
    n_bio, bio_w = bio_table.shape
    n_tok = n_b * n_l
    assert vocab <= (1 << IDX_BITS)
    assert n_pos * n_bio <= (1 << (31 - IDX_BITS))
    code = features[..., 0] * n_bio + features[..., 1]
    packed = (sentence | (code << IDX_BITS)).reshape(n_tok // 128, 128)
    pb_w = pos_w + bio_w
    pb_pad = (pb_w + 7) // 8 * 8
    pbtab = jnp.concatenate(
        [jnp.repeat(pos_table, n_bio, axis=0),
         jnp.tile(bio_table, (n_pos, 1)),
         jnp.zeros((n_pos * n_bio, pb_pad - pb_w), jnp.float32)], axis=1)
    fn = _build(n_tok, vocab, emb, n_pos * n_bio, pb_w)
    out = fn(packed, pbtab, embedding_matrix)
    return out.reshape(n_b, n_l, emb + pb_w)


# speedup vs baseline: 2.6253x; 1.4167x over previous
import functools

import jax
import jax.numpy as jnp
from jax import lax
from jax.experimental import pallas as pl
from jax.experimental.pallas import tpu as pltpu
from jax.experimental.pallas import tpu_sc as plsc

LANES = 16
NC = 2
NS = 16
NW = NC * NS
NBUF = 2
IDX_BITS = 20


@functools.lru_cache(maxsize=None)
def _build(n_tok, vocab, emb, n_pb, pb_w):
    out_w = emb + pb_w
    out_pitch = (out_w + 127) // 128 * 128
    pb_pad = (pb_w + 7) // 8 * 8
    chunk = 512
    assert n_tok % (NW * chunk * NBUF) == 0 and chunk % 128 == 0
    tok_per_w = n_tok // NW
    n_chunks = tok_per_w // chunk

    mesh = plsc.VectorSubcoreMesh(core_axis_name="c", subcore_axis_name="s")

    per_buf = [
        pltpu.VMEM((chunk // 128, 128), jnp.float32),
        pltpu.VMEM((chunk,), jnp.int32),
        pltpu.VMEM((chunk,), jnp.int32),
        pltpu.VMEM((chunk, emb), jnp.float32),
        pltpu.VMEM((chunk, pb_pad), jnp.float32),
        pltpu.SemaphoreType.DMA,
        pltpu.SemaphoreType.DMA,
        pltpu.SemaphoreType.DMA,
    ]

    @functools.partial(
        pl.kernel,
        mesh=mesh,
        out_type=jax.ShapeDtypeStruct((n_tok, out_pitch), jnp.float32),
        compiler_params=pltpu.CompilerParams(
            needs_layout_passes=False, use_tc_tiling_on_sc=False),
        scratch_types=[
            pltpu.VMEM((n_pb, pb_pad), jnp.float32),
        ] + list(per_buf) * NBUF,
    )
    def sc_kernel(packed_hbm, pbtab_hbm, table_hbm, out_hbm, pbtab_v,
                  *bufs):
        wid = lax.axis_index("s") * NC + lax.axis_index("c")
        base = wid * tok_per_w
        pltpu.sync_copy(pbtab_hbm, pbtab_v)
        iota = lax.iota(jnp.int32, LANES)
        nb = len(per_buf)

        def do_chunk(g, b, drain):
            pk_v, idx_v, code_v, word_v, pb_v, gsem, wsem, psem = (
                bufs[b * nb:(b + 1) * nb])
            cb = base + (g * NBUF + b) * chunk
            word_dst = out_hbm.at[pl.ds(cb, chunk), pl.ds(0, emb)]
            pb_dst = out_hbm.at[pl.ds(cb, chunk), pl.ds(emb, pb_pad)]

            @pl.when(drain)
            def _():
                pltpu.make_async_copy(word_v, word_dst, wsem).wait()
                pltpu.make_async_copy(pb_v, pb_dst, psem).wait()

            pltpu.sync_copy(
                packed_hbm.at[pl.ds(cb // 128, chunk // 128)], pk_v)

            def unpack_body(t, c):
                r = t // (128 // LANES)
                col = (t % (128 // LANES)) * LANES
                pk = plsc.bitcast(pk_v[r, pl.ds(col, LANES)], jnp.int32)
                idx_v[pl.ds(t * LANES, LANES)] = pk & ((1 << IDX_BITS) - 1)
                code_v[pl.ds(t * LANES, LANES)] = (
                    lax.shift_right_logical(pk, IDX_BITS))
                return c

            lax.fori_loop(0, chunk // LANES, unpack_body, 0)
            wcp = pltpu.async_copy(table_hbm.at[idx_v], word_v, gsem)

            def fill_body(t, c):
                rows = t * LANES + iota
                codes = code_v[pl.ds(t * LANES, LANES)]
                for col in range(pb_w):
                    vals = plsc.load_gather(
                        pbtab_v,
                        [codes, jnp.full((LANES,), col, jnp.int32)])
                    plsc.store_scatter(
                        pb_v, [rows, jnp.full((LANES,), col, jnp.int32)],
                        vals)
                return c

            lax.fori_loop(0, chunk // LANES, fill_body, 0)
            pltpu.async_copy(pb_v, pb_dst, psem)
            wcp.wait()
            pltpu.async_copy(word_v, word_dst, wsem)

        def pair_body(g, carry):
            for b in range(NBUF):
                do_chunk(g, b, g > 0)
            return carry

        lax.fori_loop(0, n_chunks // NBUF, pair_body, 0)
        for b in range(NBUF):
            pk_v, idx_v, code_v, word_v, pb_v, gsem, wsem, psem = (
                bufs[b * nb:(b + 1) * nb])
            cb = base + (n_chunks - NBUF + b) * chunk
            pltpu.make_async_copy(
                word_v, out_hbm.at[pl.ds(cb, chunk), pl.ds(0, emb)],
                wsem).wait()
            pltpu.make_async_copy(
                pb_v, out_hbm.at[pl.ds(cb, chunk), pl.ds(emb, pb_pad)],
                psem).wait()

    return sc_kernel


def kernel(sentence, features, embedding_matrix, pos_table, bio_table):
    n_b, n_l = sentence.shape
    vocab, emb = embedding_matrix.shape
    n_pos, pos_w = pos_table.shape
    n_bio, bio_w = bio_table.shape
    n_tok = n_b * n_l
    assert vocab <= (1 << IDX_BITS)
    assert n_pos * n_bio <= (1 << (31 - IDX_BITS))
    code = features[..., 0] * n_bio + features[..., 1]
    packed = lax.bitcast_convert_type(
        (sentence | (code << IDX_BITS)).reshape(n_tok // 128, 128),
        jnp.float32)
    pb_w = pos_w + bio_w
    pb_pad = (pb_w + 7) // 8 * 8
    pbtab = jnp.concatenate(
        [jnp.repeat(pos_table, n_bio, axis=0),
         jnp.tile(bio_table, (n_pos, 1)),
         jnp.zeros((n_pos * n_bio, pb_pad - pb_w), jnp.float32)], axis=1)
    fn = _build(n_tok, vocab, emb, n_pos * n_bio, pb_w)
    out = fn(packed, pbtab, embedding_matrix)
    return out[:, :emb + pb_w].reshape(n_b, n_l, emb + pb_w)
